# baseline (speedup 1.0000x reference)
import functools

import jax
import jax.numpy as jnp
from jax import lax
from jax.experimental import pallas as pl
from jax.experimental.pallas import tpu as pltpu
from jax.experimental.pallas import tpu_sc as plsc

N_USERS = 100000
N_ITEMS = 1000000
EMB_DIM = 32
BATCH = 16384

_NC = 2
_NS = 16
_NW = _NC * _NS
_BPW = BATCH // _NW


def _gather3(uids_hbm, iids1_hbm, iids2_hbm, uemb_hbm, iemb_hbm,
             uout_hbm, i1out_hbm, i2out_hbm,
             idx_u, idx_1, idx_2, rows_u, rows_1, rows_2,
             sem_u, sem_1, sem_2):
    wid = lax.axis_index("s") * _NC + lax.axis_index("c")
    base = wid * _BPW

    pltpu.sync_copy(uids_hbm.at[pl.ds(base, _BPW)], idx_u)
    pltpu.sync_copy(iids1_hbm.at[pl.ds(base, _BPW)], idx_1)
    pltpu.sync_copy(iids2_hbm.at[pl.ds(base, _BPW)], idx_2)

    cu = pltpu.async_copy(uemb_hbm.at[idx_u], rows_u, sem_u)
    c1 = pltpu.async_copy(iemb_hbm.at[idx_1], rows_1, sem_1)
    c2 = pltpu.async_copy(iemb_hbm.at[idx_2], rows_2, sem_2)

    cu.wait()
    pltpu.sync_copy(rows_u, uout_hbm.at[pl.ds(base, _BPW)])
    c1.wait()
    pltpu.sync_copy(rows_1, i1out_hbm.at[pl.ds(base, _BPW)])
    c2.wait()
    pltpu.sync_copy(rows_2, i2out_hbm.at[pl.ds(base, _BPW)])


@jax.jit
def _run(user_ids, item_ids_1, item_ids_2, user_emb, item_emb):
    mesh = plsc.VectorSubcoreMesh(core_axis_name="c", subcore_axis_name="s")
    f32 = jnp.float32
    call = functools.partial(
        pl.kernel,
        mesh=mesh,
        compiler_params=pltpu.CompilerParams(use_tc_tiling_on_sc=False),
        out_type=(
            jax.ShapeDtypeStruct((BATCH, EMB_DIM), f32),
            jax.ShapeDtypeStruct((BATCH, EMB_DIM), f32),
            jax.ShapeDtypeStruct((BATCH, EMB_DIM), f32),
        ),
        scratch_types=[
            pltpu.VMEM((_BPW,), jnp.int32),
            pltpu.VMEM((_BPW,), jnp.int32),
            pltpu.VMEM((_BPW,), jnp.int32),
            pltpu.VMEM((_BPW, EMB_DIM), f32),
            pltpu.VMEM((_BPW, EMB_DIM), f32),
            pltpu.VMEM((_BPW, EMB_DIM), f32),
            pltpu.SemaphoreType.DMA,
            pltpu.SemaphoreType.DMA,
            pltpu.SemaphoreType.DMA,
        ],
    )(_gather3)
    return call(user_ids.astype(jnp.int32), item_ids_1, item_ids_2,
                user_emb, item_emb)


def kernel(user_ids, item_ids_1, item_ids_2, user_emb, item_emb):
    return _run(user_ids, item_ids_1, item_ids_2, user_emb, item_emb)
